# baseline (speedup 1.0000x reference)
import functools

import jax
import jax.numpy as jnp
from jax import lax
from jax.experimental import pallas as pl
from jax.experimental.pallas import tpu as pltpu
from jax.experimental.pallas import tpu_sc as plsc

_EMB = 32
_NC, _NS = 2, 16
_NW = _NC * _NS
_CHUNK = 512
_NBUF = 4


def _sc_gather(idx_flat, table, total):
    per_w = total // _NW
    n = per_w // _CHUNK
    assert n % _NBUF == 0 and n > _NBUF
    mesh = plsc.VectorSubcoreMesh(
        core_axis_name="c", subcore_axis_name="s",
        num_cores=_NC, num_subcores=_NS)

    @functools.partial(
        pl.kernel,
        out_type=jax.ShapeDtypeStruct((total, _EMB), jnp.float32),
        mesh=mesh,
        scratch_types=[
            pltpu.VMEM((_NBUF, _CHUNK), jnp.int32),
            pltpu.VMEM((_NBUF, _CHUNK, _EMB), jnp.float32),
        ] + [pltpu.SemaphoreType.DMA] * (3 * _NBUF),
        compiler_params=pltpu.CompilerParams(use_tc_tiling_on_sc=False),
    )
    def k(idx_hbm, table_hbm, out_hbm, idx_v, rows_v, *sems):
        idx_sem = sems[0:_NBUF]
        gat_sem = sems[_NBUF:2 * _NBUF]
        st_sem = sems[2 * _NBUF:3 * _NBUF]
        wid = lax.axis_index("s") * _NC + lax.axis_index("c")
        base = wid * per_w

        def idx_desc(g, s):
            return pltpu.make_async_copy(
                idx_hbm.at[pl.ds(base + g * _CHUNK, _CHUNK)],
                idx_v.at[s], idx_sem[s])

        def gat_desc(s):
            return pltpu.make_async_copy(
                table_hbm.at[idx_v.at[s]], rows_v.at[s], gat_sem[s])

        def st_desc(g, s):
            return pltpu.make_async_copy(
                rows_v.at[s],
                out_hbm.at[pl.ds(base + g * _CHUNK, _CHUNK)],
                st_sem[s])

        for s in range(_NBUF):
            idx_desc(s, s).start()
        for g in range(_NBUF - 1):
            idx_desc(g, g).wait()
            gat_desc(g).start()

        g0 = _NBUF - 1
        idx_desc(g0, g0).wait()
        gat_desc(g0).start()
        gat_desc(0).wait()
        st_desc(0, 0).start()
        idx_desc(_NBUF, 0).start()

        @pl.loop(0, (n - _NBUF) // _NBUF)
        def main(t):
            for b in range(_NBUF):
                g = _NBUF + _NBUF * t + b
                s = b
                so = (b + 1) % _NBUF
                st_desc(g - _NBUF, s).wait()
                idx_desc(g, s).wait()
                gat_desc(s).start()
                gat_desc(so).wait()
                st_desc(g - (_NBUF - 1), so).start()
                idx_desc(g + 1, so).start()

        for go in range(n - _NBUF + 1, n):
            so = go % _NBUF
            gat_desc(so).wait()
            st_desc(go, so).start()
        for c in range(n - _NBUF, n):
            st_desc(c, c % _NBUF).wait()
        idx_desc(n, n % _NBUF).wait()

    return k(idx_flat, table)


def kernel(x, W):
    b, h = x.shape
    total = b * h
    idx_flat = x.astype(jnp.int32).reshape(total)
    idx_flat = jnp.pad(idx_flat, (0, _CHUNK))
    out = _sc_gather(idx_flat, W, total)
    return out.reshape(b, h, _EMB)
